# baseline (speedup 1.0000x reference)
import functools

import jax
import jax.numpy as jnp
import numpy as np
from jax import lax
from jax.experimental import pallas as pl
from jax.experimental.pallas import tpu as pltpu
from jax.experimental.pallas import tpu_sc as plsc

ENC = 8
SIG = 0.1
BM = 1024
_SH = BM.bit_length() - 1
GBM = 512


def _pos_enc(x_blk):
    t = jax.lax.broadcasted_iota(jnp.int32, (1, ENC // 2), 1).astype(jnp.float32) / ENC
    fr = (2.0 * np.pi) * jnp.exp(t * float(np.log(SIG)))
    x0 = x_blk[:, 0:1]
    x1 = x_blk[:, 1:2]
    return jnp.concatenate(
        [jnp.sin(x0 * fr), jnp.cos(x0 * fr), jnp.sin(x1 * fr), jnp.cos(x1 * fr)],
        axis=1,
    )


def _gating_body(xt_ref, x_ref, gW0_ref, gb0_ref, gW1_ref, gb1_ref,
                 i1_ref, i2_ref, g1_ref, g2_ref, pe_ref, h_ref,
                 *, n_experts):
    xT = xt_ref[...]
    bm = xT.shape[1]
    nf = ENC // 2
    frc = (2.0 * np.pi) * jnp.exp(
        jax.lax.broadcasted_iota(jnp.int32, (nf, 1), 0).astype(jnp.float32)
        / ENC * float(np.log(SIG)))
    phA = frc * xT[0:1, :]
    phB = frc * xT[1:2, :]
    peT = jnp.concatenate(
        [jnp.sin(phA), jnp.cos(phA), jnp.sin(phB), jnp.cos(phB)],
        axis=0)
    hT = jax.nn.relu(
        jax.lax.dot_general(gW0_ref[...], peT, (((0,), (0,)), ((), ())),
                            preferred_element_type=jnp.float32)
        + gb0_ref[...][:, None])
    logitsT = (jax.lax.dot_general(gW1_ref[...], hT, (((0,), (0,)), ((), ())),
                                   preferred_element_type=jnp.float32)
               + gb1_ref[...][:, None])
    idxs = jax.lax.broadcasted_iota(jnp.int32, (n_experts, bm), 0)
    m1 = jnp.max(logitsT, axis=0, keepdims=True)
    i1 = jnp.min(jnp.where(logitsT == m1, idxs, n_experts), axis=0,
                 keepdims=True)
    masked = jnp.where(idxs == i1, -jnp.inf, logitsT)
    m2 = jnp.max(masked, axis=0, keepdims=True)
    i2 = jnp.min(jnp.where(masked == m2, idxs, n_experts), axis=0,
                 keepdims=True)
    r = jnp.exp(m2 - m1)
    i1_ref[...] = i1[0]
    i2_ref[...] = i2[0]
    g1_ref[...] = (1.0 / (1.0 + r))[0]
    g2_ref[...] = (r / (1.0 + r))[0]
    pe = _pos_enc(x_ref[...])
    pe_ref[...] = jnp.concatenate(
        [pe, jnp.zeros((bm, 128 - 2 * ENC), jnp.float32)], axis=1)
    o1 = (i1 == idxs).astype(jnp.int32)
    o2 = (i2 == idxs).astype(jnp.int32)
    rows = []
    for hh in range(bm // 256):
        sl = slice(hh * 256, (hh + 1) * 256)
        rows.append(jnp.sum(o1[:, sl], axis=1)[None])
        rows.append(jnp.sum(o2[:, sl], axis=1)[None])
    h_ref[...] = jnp.concatenate(rows, axis=0)[None]


def _gating(x, gW0, gb0, gW1, gb1, n, n_experts):
    nblk = n // GBM
    nh = 2 * (GBM // 256)
    full = lambda a: pl.BlockSpec(a.shape, lambda m: (0,) * a.ndim)
    return pl.pallas_call(
        functools.partial(_gating_body, n_experts=n_experts),
        grid=(nblk,),
        in_specs=[
            pl.BlockSpec((2, GBM), lambda m: (0, m)),
            pl.BlockSpec((GBM, 2), lambda m: (m, 0)),
            full(gW0), full(gb0), full(gW1), full(gb1),
        ],
        out_specs=[
            pl.BlockSpec((GBM,), lambda m: (m,)),
            pl.BlockSpec((GBM,), lambda m: (m,)),
            pl.BlockSpec((GBM,), lambda m: (m,)),
            pl.BlockSpec((GBM,), lambda m: (m,)),
            pl.BlockSpec((GBM, 128), lambda m: (m, 0)),
            pl.BlockSpec((1, nh, n_experts), lambda m: (m, 0, 0)),
        ],
        out_shape=[
            jax.ShapeDtypeStruct((n,), jnp.int32),
            jax.ShapeDtypeStruct((n,), jnp.int32),
            jax.ShapeDtypeStruct((n,), jnp.float32),
            jax.ShapeDtypeStruct((n,), jnp.float32),
            jax.ShapeDtypeStruct((n, 128), jnp.float32),
            jax.ShapeDtypeStruct((nblk, nh, n_experts), jnp.int32),
        ],
        compiler_params=pltpu.CompilerParams(
            dimension_semantics=("arbitrary",),
        ),
    )(x.T, x, gW0, gb0, gW1, gb1)


def _dispatch(i1, i2, g1, g2, pe, hist, *, n, n_experts, pmax, nbp):
    pairs = 2 * n
    nw = 32
    chunk = pairs // nw
    nvec = chunk // 16
    mesh = plsc.VectorSubcoreMesh(core_axis_name="c", subcore_axis_name="s")

    @functools.partial(
        pl.kernel,
        out_type=(
            jax.ShapeDtypeStruct((pmax, 128), jnp.float32),
            jax.ShapeDtypeStruct((pairs // 128, 128), jnp.int32),
            jax.ShapeDtypeStruct((nbp,), jnp.int32),
        ),
        mesh=mesh,
        scratch_types=[
            pltpu.VMEM((chunk,), jnp.int32),
            pltpu.VMEM((chunk,), jnp.float32),
            pltpu.VMEM((nw * n_experts,), jnp.int32),
            pltpu.VMEM((chunk, 128), jnp.float32),
            pltpu.VMEM((2, 128), jnp.int32),
            pltpu.VMEM((nbp,), jnp.int32),
            pltpu.SemaphoreType.DMA,
        ],
        compiler_params=pltpu.CompilerParams(needs_layout_passes=False),
    )
    def dispatch(i1_hbm, i2_hbm, g1_hbm, g2_hbm, pe_hbm, hist_hbm,
                 xpad_hbm, pos_hbm, be_hbm,
                 ids_v, g_v, hist_v, bufg_v, pos_v, be_v, sem):
        wid = lax.axis_index("c") * 16 + lax.axis_index("s")
        s = wid % 2
        mb = wid // 2
        tok0 = mb * chunk

        @pl.when(s == 0)
        def _():
            pltpu.sync_copy(i1_hbm.at[pl.ds(tok0, chunk)], ids_v)
            pltpu.sync_copy(g1_hbm.at[pl.ds(tok0, chunk)], g_v)

        @pl.when(s == 1)
        def _():
            pltpu.sync_copy(i2_hbm.at[pl.ds(tok0, chunk)], ids_v)
            pltpu.sync_copy(g2_hbm.at[pl.ds(tok0, chunk)], g_v)

        pltpu.sync_copy(pe_hbm.at[pl.ds(tok0, chunk)], bufg_v)
        pltpu.sync_copy(hist_hbm, hist_v)

        iota = lax.iota(jnp.int32, 16)
        base = []
        po_next = []
        po_acc = jnp.int32(0)
        for e in range(n_experts):
            h0 = plsc.load_gather(hist_v, [iota * n_experts + e])
            h1 = plsc.load_gather(hist_v, [(iota + 16) * n_experts + e])
            tc_e = jnp.sum(h0) + jnp.sum(h1)
            prior = (jnp.sum(jnp.where(iota < wid, h0, 0))
                     + jnp.sum(jnp.where(iota + 16 < wid, h1, 0)))
            pc_e = jnp.left_shift(jnp.right_shift(tc_e + BM - 1, _SH), _SH)
            base.append(po_acc + prior)
            po_acc = po_acc + pc_e
            po_next.append(po_acc)

        @pl.when(wid == 0)
        def _():
            for i in range(nbp // 16):
                bi = iota + 16 * i
                bev = jnp.zeros((16,), jnp.int32)
                for e in range(n_experts):
                    nb_e = jnp.right_shift(po_next[e], _SH)
                    bev = bev + (bi >= nb_e).astype(jnp.int32)
                be_v[pl.ds(16 * i, 16)] = jnp.minimum(bev, n_experts - 1)
            pltpu.sync_copy(be_v, be_hbm)

        rb = list(base)
        zeros16 = jnp.zeros((16,), jnp.int32)
        for i in range(nvec):
            ev = ids_v[pl.ds(16 * i, 16)]
            gv = g_v[pl.ds(16 * i, 16)]
            pos = zeros16
            for e in range(n_experts):
                m = ev == e
                mi = m.astype(jnp.int32)
                cs = plsc.cumsum(mi)
                pos = jnp.where(m, rb[e] + cs - mi, pos)
                rb[e] = rb[e] + jnp.sum(mi)
            pos_v[i // 8, pl.ds((i % 8) * 16, 16)] = pos
            plsc.store_scatter(bufg_v, [iota + 16 * i, zeros16 + 16], gv)

        for h in range(2):
            src = bufg_v.at[pl.ds(h * 128, 128)]
            pltpu.async_copy(src, xpad_hbm.at[pos_v.at[h]], sem).wait()
        pltpu.sync_copy(pos_v, pos_hbm.at[pl.ds(s * (n // 128) + mb * 2, 2)])

    return dispatch(i1, i2, g1, g2, pe, hist)


def _group_body(be_ref, x_ref,
                SW0_ref, Sb0_ref, SW1_ref, Sb1_ref, SW2_ref, Sb2_ref,
                AW0_ref, Ab0_ref, AW1_ref, Ab1_ref, AW2_ref, Ab2_ref,
                out_ref):
    b = pl.program_id(0)
    e = be_ref[b]
    xg = x_ref[...]
    pe = xg[:, 0:16]
    gate = xg[:, 16:17]
    sw0 = SW0_ref[pl.ds(e, 1)][0]
    sb0 = Sb0_ref[pl.ds(e, 1)][0]
    sw1 = SW1_ref[pl.ds(e, 1)][0]
    sb1 = Sb1_ref[pl.ds(e, 1)][0]
    sw2 = SW2_ref[pl.ds(e, 1)][0]
    sb2 = Sb2_ref[pl.ds(e, 1)][0]
    aw0 = AW0_ref[pl.ds(e, 1)][0]
    ab0 = Ab0_ref[pl.ds(e, 1)][0]
    aw1 = AW1_ref[pl.ds(e, 1)][0]
    ab1 = Ab1_ref[pl.ds(e, 1)][0]
    aw2 = AW2_ref[0]
    ab2 = Ab2_ref[pl.ds(e, 1)][0]
    s = jax.nn.relu(jnp.dot(pe, sw0, preferred_element_type=jnp.float32) + sb0)
    s = jax.nn.relu(jnp.dot(s, sw1, preferred_element_type=jnp.float32) + sb1)
    iso = jnp.dot(s, sw2, preferred_element_type=jnp.float32) + sb2
    a = jax.nn.relu(iso * aw0[0][None, :] + ab0)
    a = jax.nn.relu(jnp.dot(a, aw1, preferred_element_type=jnp.float32) + ab1)
    o = jnp.dot(a, aw2, preferred_element_type=jnp.float32) + ab2
    out_ref[...] = gate * o


def _grouped(be, xpad, SW0, Sb0, SW1, Sb1, SW2, Sb2,
             AW0, Ab0, AW1, Ab1, AW2, Ab2, *, pmax, g_out):
    nb = pmax // BM
    Sb0, Sb1, Sb2 = Sb0[:, None], Sb1[:, None], Sb2[:, None]
    Ab0, Ab1, Ab2 = Ab0[:, None], Ab1[:, None], Ab2[:, None]
    wspec = lambda a: pl.BlockSpec(a.shape, lambda m, be: (0,) * a.ndim)
    aw2spec = pl.BlockSpec((1,) + AW2.shape[1:],
                           lambda m, be: (be[m], 0, 0))
    return pl.pallas_call(
        _group_body,
        grid_spec=pltpu.PrefetchScalarGridSpec(
            num_scalar_prefetch=1,
            grid=(nb,),
            in_specs=[
                pl.BlockSpec((BM, 128), lambda m, be: (m, 0)),
                wspec(SW0), wspec(Sb0), wspec(SW1), wspec(Sb1),
                wspec(SW2), wspec(Sb2),
                wspec(AW0), wspec(Ab0), wspec(AW1), wspec(Ab1),
                aw2spec, wspec(Ab2),
            ],
            out_specs=pl.BlockSpec((BM, g_out), lambda m, be: (m, 0)),
        ),
        out_shape=jax.ShapeDtypeStruct((pmax, g_out), jnp.float32),
        compiler_params=pltpu.CompilerParams(
            dimension_semantics=("arbitrary",),
        ),
    )(be, xpad, SW0, Sb0, SW1, Sb1, SW2, Sb2,
      AW0, Ab0, AW1, Ab1, AW2, Ab2)


def _combine(outpad, pos, *, n, g_out):
    nw = 32
    tpw = n // nw
    sub = 16
    nsub = tpw // sub
    mesh = plsc.VectorSubcoreMesh(core_axis_name="c", subcore_axis_name="s")

    @functools.partial(
        pl.kernel,
        out_type=jax.ShapeDtypeStruct((n, g_out), jnp.float32),
        mesh=mesh,
        scratch_types=[
            pltpu.VMEM((tpw,), jnp.int32),
            pltpu.VMEM((tpw,), jnp.int32),
            pltpu.VMEM((sub, g_out), jnp.float32),
            pltpu.VMEM((sub, g_out), jnp.float32),
            pltpu.VMEM((sub, g_out), jnp.float32),
            pltpu.VMEM((sub, g_out), jnp.float32),
            pltpu.SemaphoreType.DMA,
            pltpu.SemaphoreType.DMA,
            pltpu.SemaphoreType.DMA,
            pltpu.SemaphoreType.DMA,
        ],
        compiler_params=pltpu.CompilerParams(needs_layout_passes=False),
    )
    def combine(outpad_hbm, pos_hbm, y_hbm, p1_v, p2_v,
                rA1, rA2, rB1, rB2, sA, sB, swA, swB):
        wid = lax.axis_index("c") * 16 + lax.axis_index("s")
        pltpu.sync_copy(pos_hbm.at[wid], p1_v)
        pltpu.sync_copy(pos_hbm.at[nw + wid], p2_v)
        bufs = [(rA1, rA2, sA, swA), (rB1, rB2, sB, swB)]

        def gathers(k, i):
            r1b, r2b, sem, _ = bufs[i]
            c1 = pltpu.async_copy(
                outpad_hbm.at[p1_v.at[pl.ds(k * sub, sub)]], r1b, sem)
            c2 = pltpu.async_copy(
                outpad_hbm.at[p2_v.at[pl.ds(k * sub, sub)]], r2b, sem)
            return c1, c2

        pend = [None, None]
        wb = [None, None]
        pend[0] = gathers(0, 0)
        for k in range(nsub):
            i = k % 2
            j = (k + 1) % 2
            if k + 1 < nsub:
                if wb[j] is not None:
                    wb[j].wait()
                    wb[j] = None
                pend[j] = gathers(k + 1, j)
            c1, c2 = pend[i]
            c1.wait()
            c2.wait()
            r1b, r2b = bufs[i][0], bufs[i][1]

            @plsc.parallel_loop(0, sub * g_out, 64, unroll=4)
            def _(kk):
                r = kk // g_out
                o = kk % g_out
                for u in range(4):
                    plsc.addupdate(r1b.at[r, pl.ds(o + u * 16, 16)],
                                   r2b[r, pl.ds(o + u * 16, 16)])

            wb[i] = pltpu.async_copy(
                r1b, y_hbm.at[pl.ds(wid * tpw + k * sub, sub)], bufs[i][3])
        for i in range(2):
            if wb[i] is not None:
                wb[i].wait()

    return combine(outpad, pos)


def kernel(x, gW0, gb0, gW1, gb1, SW0, Sb0, SW1, Sb1, SW2, Sb2,
           AW0, Ab0, AW1, Ab1, AW2, Ab2):
    n = x.shape[0]
    n_experts = SW0.shape[0]
    g_out = AW2.shape[2]
    pmax = 2 * n + n_experts * BM
    nbp = ((pmax // BM) + 15) // 16 * 16

    i1, i2, g1, g2, pe, hist = _gating(x, gW0, gb0, gW1, gb1, n, n_experts)

    xpad, pos, be = _dispatch(i1, i2, g1, g2, pe, hist.reshape(-1),
                              n=n, n_experts=n_experts,
                              pmax=pmax, nbp=nbp)
    outpad = _grouped(be, xpad, SW0, Sb0, SW1, Sb1, SW2, Sb2,
                      AW0, Ab0, AW1, Ab1, AW2, Ab2,
                      pmax=pmax, g_out=g_out)
    y = _combine(outpad, pos, n=n, g_out=g_out)
    return y


# speedup vs baseline: 1.0300x; 1.0300x over previous
import functools

import jax
import jax.numpy as jnp
import numpy as np
from jax import lax
from jax.experimental import pallas as pl
from jax.experimental.pallas import tpu as pltpu
from jax.experimental.pallas import tpu_sc as plsc

ENC = 8
SIG = 0.1
BM = 512
_SH = BM.bit_length() - 1
GBM = 512


def _pos_enc(x_blk):
    t = jax.lax.broadcasted_iota(jnp.int32, (1, ENC // 2), 1).astype(jnp.float32) / ENC
    fr = (2.0 * np.pi) * jnp.exp(t * float(np.log(SIG)))
    x0 = x_blk[:, 0:1]
    x1 = x_blk[:, 1:2]
    return jnp.concatenate(
        [jnp.sin(x0 * fr), jnp.cos(x0 * fr), jnp.sin(x1 * fr), jnp.cos(x1 * fr)],
        axis=1,
    )


def _gating_body(xt_ref, x_ref, gW0_ref, gb0_ref, gW1_ref, gb1_ref,
                 i1_ref, i2_ref, g1_ref, g2_ref, pe_ref, h_ref,
                 *, n_experts):
    xT = xt_ref[...]
    bm = xT.shape[1]
    nf = ENC // 2
    frc = (2.0 * np.pi) * jnp.exp(
        jax.lax.broadcasted_iota(jnp.int32, (nf, 1), 0).astype(jnp.float32)
        / ENC * float(np.log(SIG)))
    phA = frc * xT[0:1, :]
    phB = frc * xT[1:2, :]
    peT = jnp.concatenate(
        [jnp.sin(phA), jnp.cos(phA), jnp.sin(phB), jnp.cos(phB)],
        axis=0)
    hT = jax.nn.relu(
        jax.lax.dot_general(gW0_ref[...], peT, (((0,), (0,)), ((), ())),
                            preferred_element_type=jnp.float32)
        + gb0_ref[...][:, None])
    logitsT = (jax.lax.dot_general(gW1_ref[...], hT, (((0,), (0,)), ((), ())),
                                   preferred_element_type=jnp.float32)
               + gb1_ref[...][:, None])
    idxs = jax.lax.broadcasted_iota(jnp.int32, (n_experts, bm), 0)
    m1 = jnp.max(logitsT, axis=0, keepdims=True)
    i1 = jnp.min(jnp.where(logitsT == m1, idxs, n_experts), axis=0,
                 keepdims=True)
    masked = jnp.where(idxs == i1, -jnp.inf, logitsT)
    m2 = jnp.max(masked, axis=0, keepdims=True)
    i2 = jnp.min(jnp.where(masked == m2, idxs, n_experts), axis=0,
                 keepdims=True)
    r = jnp.exp(m2 - m1)
    i1_ref[...] = i1[0]
    i2_ref[...] = i2[0]
    g1_ref[...] = (1.0 / (1.0 + r))[0]
    g2_ref[...] = (r / (1.0 + r))[0]
    pe = _pos_enc(x_ref[...])
    pe_ref[...] = jnp.concatenate(
        [pe, jnp.zeros((bm, 128 - 2 * ENC), jnp.float32)], axis=1)
    o1 = (i1 == idxs).astype(jnp.int32)
    o2 = (i2 == idxs).astype(jnp.int32)
    rows = []
    for hh in range(bm // 256):
        sl = slice(hh * 256, (hh + 1) * 256)
        rows.append(jnp.sum(o1[:, sl], axis=1)[None])
        rows.append(jnp.sum(o2[:, sl], axis=1)[None])
    h_ref[...] = jnp.concatenate(rows, axis=0)[None]


def _gating(x, gW0, gb0, gW1, gb1, n, n_experts):
    nblk = n // GBM
    nh = 2 * (GBM // 256)
    full = lambda a: pl.BlockSpec(a.shape, lambda m: (0,) * a.ndim)
    return pl.pallas_call(
        functools.partial(_gating_body, n_experts=n_experts),
        grid=(nblk,),
        in_specs=[
            pl.BlockSpec((2, GBM), lambda m: (0, m)),
            pl.BlockSpec((GBM, 2), lambda m: (m, 0)),
            full(gW0), full(gb0), full(gW1), full(gb1),
        ],
        out_specs=[
            pl.BlockSpec((GBM,), lambda m: (m,)),
            pl.BlockSpec((GBM,), lambda m: (m,)),
            pl.BlockSpec((GBM,), lambda m: (m,)),
            pl.BlockSpec((GBM,), lambda m: (m,)),
            pl.BlockSpec((GBM, 128), lambda m: (m, 0)),
            pl.BlockSpec((1, nh, n_experts), lambda m: (m, 0, 0)),
        ],
        out_shape=[
            jax.ShapeDtypeStruct((n,), jnp.int32),
            jax.ShapeDtypeStruct((n,), jnp.int32),
            jax.ShapeDtypeStruct((n,), jnp.float32),
            jax.ShapeDtypeStruct((n,), jnp.float32),
            jax.ShapeDtypeStruct((n, 128), jnp.float32),
            jax.ShapeDtypeStruct((nblk, nh, n_experts), jnp.int32),
        ],
        compiler_params=pltpu.CompilerParams(
            dimension_semantics=("arbitrary",),
        ),
    )(x.T, x, gW0, gb0, gW1, gb1)


def _dispatch(i1, i2, g1, g2, pe, hist, *, n, n_experts, pmax, nbp):
    pairs = 2 * n
    nw = 32
    chunk = pairs // nw
    nvec = chunk // 16
    mesh = plsc.VectorSubcoreMesh(core_axis_name="c", subcore_axis_name="s")

    @functools.partial(
        pl.kernel,
        out_type=(
            jax.ShapeDtypeStruct((pmax, 128), jnp.float32),
            jax.ShapeDtypeStruct((pairs // 128, 128), jnp.int32),
            jax.ShapeDtypeStruct((nbp,), jnp.int32),
        ),
        mesh=mesh,
        scratch_types=[
            pltpu.VMEM((chunk,), jnp.int32),
            pltpu.VMEM((chunk,), jnp.float32),
            pltpu.VMEM((nw * n_experts,), jnp.int32),
            pltpu.VMEM((chunk, 128), jnp.float32),
            pltpu.VMEM((2, 128), jnp.int32),
            pltpu.VMEM((nbp,), jnp.int32),
            pltpu.SemaphoreType.DMA,
        ],
        compiler_params=pltpu.CompilerParams(needs_layout_passes=False),
    )
    def dispatch(i1_hbm, i2_hbm, g1_hbm, g2_hbm, pe_hbm, hist_hbm,
                 xpad_hbm, pos_hbm, be_hbm,
                 ids_v, g_v, hist_v, bufg_v, pos_v, be_v, sem):
        wid = lax.axis_index("c") * 16 + lax.axis_index("s")
        s = wid % 2
        mb = wid // 2
        tok0 = mb * chunk

        @pl.when(s == 0)
        def _():
            pltpu.sync_copy(i1_hbm.at[pl.ds(tok0, chunk)], ids_v)
            pltpu.sync_copy(g1_hbm.at[pl.ds(tok0, chunk)], g_v)

        @pl.when(s == 1)
        def _():
            pltpu.sync_copy(i2_hbm.at[pl.ds(tok0, chunk)], ids_v)
            pltpu.sync_copy(g2_hbm.at[pl.ds(tok0, chunk)], g_v)

        pltpu.sync_copy(pe_hbm.at[pl.ds(tok0, chunk)], bufg_v)
        pltpu.sync_copy(hist_hbm, hist_v)

        iota = lax.iota(jnp.int32, 16)
        base = []
        po_next = []
        po_acc = jnp.int32(0)
        for e in range(n_experts):
            h0 = plsc.load_gather(hist_v, [iota * n_experts + e])
            h1 = plsc.load_gather(hist_v, [(iota + 16) * n_experts + e])
            tc_e = jnp.sum(h0) + jnp.sum(h1)
            prior = (jnp.sum(jnp.where(iota < wid, h0, 0))
                     + jnp.sum(jnp.where(iota + 16 < wid, h1, 0)))
            pc_e = jnp.left_shift(jnp.right_shift(tc_e + BM - 1, _SH), _SH)
            base.append(po_acc + prior)
            po_acc = po_acc + pc_e
            po_next.append(po_acc)

        @pl.when(wid == 0)
        def _():
            for i in range(nbp // 16):
                bi = iota + 16 * i
                bev = jnp.zeros((16,), jnp.int32)
                for e in range(n_experts):
                    nb_e = jnp.right_shift(po_next[e], _SH)
                    bev = bev + (bi >= nb_e).astype(jnp.int32)
                be_v[pl.ds(16 * i, 16)] = jnp.minimum(bev, n_experts - 1)
            pltpu.sync_copy(be_v, be_hbm)

        rb = list(base)
        zeros16 = jnp.zeros((16,), jnp.int32)
        for i in range(nvec):
            ev = ids_v[pl.ds(16 * i, 16)]
            gv = g_v[pl.ds(16 * i, 16)]
            pos = zeros16
            for e in range(n_experts):
                m = ev == e
                mi = m.astype(jnp.int32)
                cs = plsc.cumsum(mi)
                pos = jnp.where(m, rb[e] + cs - mi, pos)
                rb[e] = rb[e] + jnp.sum(mi)
            pos_v[i // 8, pl.ds((i % 8) * 16, 16)] = pos
            plsc.store_scatter(bufg_v, [iota + 16 * i, zeros16 + 16], gv)

        for h in range(2):
            src = bufg_v.at[pl.ds(h * 128, 128)]
            pltpu.async_copy(src, xpad_hbm.at[pos_v.at[h]], sem).wait()
        pltpu.sync_copy(pos_v, pos_hbm.at[pl.ds(s * (n // 128) + mb * 2, 2)])

    return dispatch(i1, i2, g1, g2, pe, hist)


def _group_body(be_ref, x_ref,
                SW0_ref, Sb0_ref, SW1_ref, Sb1_ref, SW2_ref, Sb2_ref,
                AW0_ref, Ab0_ref, AW1_ref, Ab1_ref, AW2_ref, Ab2_ref,
                out_ref):
    b = pl.program_id(0)
    e = be_ref[b]
    xg = x_ref[...]
    pe = xg[:, 0:16]
    gate = xg[:, 16:17]
    sw0 = SW0_ref[pl.ds(e, 1)][0]
    sb0 = Sb0_ref[pl.ds(e, 1)][0]
    sw1 = SW1_ref[pl.ds(e, 1)][0]
    sb1 = Sb1_ref[pl.ds(e, 1)][0]
    sw2 = SW2_ref[pl.ds(e, 1)][0]
    sb2 = Sb2_ref[pl.ds(e, 1)][0]
    aw0 = AW0_ref[pl.ds(e, 1)][0]
    ab0 = Ab0_ref[pl.ds(e, 1)][0]
    aw1 = AW1_ref[pl.ds(e, 1)][0]
    ab1 = Ab1_ref[pl.ds(e, 1)][0]
    aw2 = AW2_ref[0]
    ab2 = Ab2_ref[pl.ds(e, 1)][0]
    s = jax.nn.relu(jnp.dot(pe, sw0, preferred_element_type=jnp.float32) + sb0)
    s = jax.nn.relu(jnp.dot(s, sw1, preferred_element_type=jnp.float32) + sb1)
    iso = jnp.dot(s, sw2, preferred_element_type=jnp.float32) + sb2
    a = jax.nn.relu(iso * aw0[0][None, :] + ab0)
    a = jax.nn.relu(jnp.dot(a, aw1, preferred_element_type=jnp.float32) + ab1)
    o = jnp.dot(a, aw2, preferred_element_type=jnp.float32) + ab2
    out_ref[...] = gate * o


def _grouped(be, xpad, SW0, Sb0, SW1, Sb1, SW2, Sb2,
             AW0, Ab0, AW1, Ab1, AW2, Ab2, *, pmax, g_out):
    nb = pmax // BM
    Sb0, Sb1, Sb2 = Sb0[:, None], Sb1[:, None], Sb2[:, None]
    Ab0, Ab1, Ab2 = Ab0[:, None], Ab1[:, None], Ab2[:, None]
    wspec = lambda a: pl.BlockSpec(a.shape, lambda m, be: (0,) * a.ndim)
    aw2spec = pl.BlockSpec((1,) + AW2.shape[1:],
                           lambda m, be: (be[m], 0, 0))
    return pl.pallas_call(
        _group_body,
        grid_spec=pltpu.PrefetchScalarGridSpec(
            num_scalar_prefetch=1,
            grid=(nb,),
            in_specs=[
                pl.BlockSpec((BM, 128), lambda m, be: (m, 0)),
                wspec(SW0), wspec(Sb0), wspec(SW1), wspec(Sb1),
                wspec(SW2), wspec(Sb2),
                wspec(AW0), wspec(Ab0), wspec(AW1), wspec(Ab1),
                aw2spec, wspec(Ab2),
            ],
            out_specs=pl.BlockSpec((BM, g_out), lambda m, be: (m, 0)),
        ),
        out_shape=jax.ShapeDtypeStruct((pmax, g_out), jnp.float32),
        compiler_params=pltpu.CompilerParams(
            dimension_semantics=("arbitrary",),
        ),
    )(be, xpad, SW0, Sb0, SW1, Sb1, SW2, Sb2,
      AW0, Ab0, AW1, Ab1, AW2, Ab2)


def _combine(outpad, pos, *, n, g_out):
    nw = 32
    tpw = n // nw
    sub = 16
    nsub = tpw // sub
    mesh = plsc.VectorSubcoreMesh(core_axis_name="c", subcore_axis_name="s")

    @functools.partial(
        pl.kernel,
        out_type=jax.ShapeDtypeStruct((n, g_out), jnp.float32),
        mesh=mesh,
        scratch_types=[
            pltpu.VMEM((tpw,), jnp.int32),
            pltpu.VMEM((tpw,), jnp.int32),
            pltpu.VMEM((sub, g_out), jnp.float32),
            pltpu.VMEM((sub, g_out), jnp.float32),
            pltpu.VMEM((sub, g_out), jnp.float32),
            pltpu.VMEM((sub, g_out), jnp.float32),
            pltpu.SemaphoreType.DMA,
            pltpu.SemaphoreType.DMA,
            pltpu.SemaphoreType.DMA,
            pltpu.SemaphoreType.DMA,
        ],
        compiler_params=pltpu.CompilerParams(needs_layout_passes=False),
    )
    def combine(outpad_hbm, pos_hbm, y_hbm, p1_v, p2_v,
                rA1, rA2, rB1, rB2, sA, sB, swA, swB):
        wid = lax.axis_index("c") * 16 + lax.axis_index("s")
        pltpu.sync_copy(pos_hbm.at[wid], p1_v)
        pltpu.sync_copy(pos_hbm.at[nw + wid], p2_v)
        bufs = [(rA1, rA2, sA, swA), (rB1, rB2, sB, swB)]

        def gathers(k, i):
            r1b, r2b, sem, _ = bufs[i]
            c1 = pltpu.async_copy(
                outpad_hbm.at[p1_v.at[pl.ds(k * sub, sub)]], r1b, sem)
            c2 = pltpu.async_copy(
                outpad_hbm.at[p2_v.at[pl.ds(k * sub, sub)]], r2b, sem)
            return c1, c2

        pend = [None, None]
        wb = [None, None]
        pend[0] = gathers(0, 0)
        for k in range(nsub):
            i = k % 2
            j = (k + 1) % 2
            if k + 1 < nsub:
                if wb[j] is not None:
                    wb[j].wait()
                    wb[j] = None
                pend[j] = gathers(k + 1, j)
            c1, c2 = pend[i]
            c1.wait()
            c2.wait()
            r1b, r2b = bufs[i][0], bufs[i][1]

            @plsc.parallel_loop(0, sub * g_out, 64, unroll=4)
            def _(kk):
                r = kk // g_out
                o = kk % g_out
                for u in range(4):
                    plsc.addupdate(r1b.at[r, pl.ds(o + u * 16, 16)],
                                   r2b[r, pl.ds(o + u * 16, 16)])

            wb[i] = pltpu.async_copy(
                r1b, y_hbm.at[pl.ds(wid * tpw + k * sub, sub)], bufs[i][3])
        for i in range(2):
            if wb[i] is not None:
                wb[i].wait()

    return combine(outpad, pos)


def kernel(x, gW0, gb0, gW1, gb1, SW0, Sb0, SW1, Sb1, SW2, Sb2,
           AW0, Ab0, AW1, Ab1, AW2, Ab2):
    n = x.shape[0]
    n_experts = SW0.shape[0]
    g_out = AW2.shape[2]
    pmax = 2 * n + n_experts * BM
    nbp = ((pmax // BM) + 15) // 16 * 16

    i1, i2, g1, g2, pe, hist = _gating(x, gW0, gb0, gW1, gb1, n, n_experts)

    xpad, pos, be = _dispatch(i1, i2, g1, g2, pe, hist.reshape(-1),
                              n=n, n_experts=n_experts,
                              pmax=pmax, nbp=nbp)
    outpad = _grouped(be, xpad, SW0, Sb0, SW1, Sb1, SW2, Sb2,
                      AW0, Ab0, AW1, Ab1, AW2, Ab2,
                      pmax=pmax, g_out=g_out)
    y = _combine(outpad, pos, n=n, g_out=g_out)
    return y


# speedup vs baseline: 1.0366x; 1.0065x over previous
import functools

import jax
import jax.numpy as jnp
import numpy as np
from jax import lax
from jax.experimental import pallas as pl
from jax.experimental.pallas import tpu as pltpu
from jax.experimental.pallas import tpu_sc as plsc

ENC = 8
SIG = 0.1
BM = 512
_SH = BM.bit_length() - 1
GBM = 1024


def _pos_enc(x_blk):
    t = jax.lax.broadcasted_iota(jnp.int32, (1, ENC // 2), 1).astype(jnp.float32) / ENC
    fr = (2.0 * np.pi) * jnp.exp(t * float(np.log(SIG)))
    x0 = x_blk[:, 0:1]
    x1 = x_blk[:, 1:2]
    return jnp.concatenate(
        [jnp.sin(x0 * fr), jnp.cos(x0 * fr), jnp.sin(x1 * fr), jnp.cos(x1 * fr)],
        axis=1,
    )


def _gating_body(xt_ref, x_ref, gW0_ref, gb0_ref, gW1_ref, gb1_ref,
                 i1_ref, i2_ref, g1_ref, g2_ref, pe_ref, h_ref,
                 *, n_experts):
    xT = xt_ref[...]
    bm = xT.shape[1]
    nf = ENC // 2
    frc = (2.0 * np.pi) * jnp.exp(
        jax.lax.broadcasted_iota(jnp.int32, (nf, 1), 0).astype(jnp.float32)
        / ENC * float(np.log(SIG)))
    phA = frc * xT[0:1, :]
    phB = frc * xT[1:2, :]
    peT = jnp.concatenate(
        [jnp.sin(phA), jnp.cos(phA), jnp.sin(phB), jnp.cos(phB)],
        axis=0)
    hT = jax.nn.relu(
        jax.lax.dot_general(gW0_ref[...], peT, (((0,), (0,)), ((), ())),
                            preferred_element_type=jnp.float32)
        + gb0_ref[...][:, None])
    logitsT = (jax.lax.dot_general(gW1_ref[...], hT, (((0,), (0,)), ((), ())),
                                   preferred_element_type=jnp.float32)
               + gb1_ref[...][:, None])
    idxs = jax.lax.broadcasted_iota(jnp.int32, (n_experts, bm), 0)
    m1 = jnp.max(logitsT, axis=0, keepdims=True)
    i1 = jnp.min(jnp.where(logitsT == m1, idxs, n_experts), axis=0,
                 keepdims=True)
    masked = jnp.where(idxs == i1, -jnp.inf, logitsT)
    m2 = jnp.max(masked, axis=0, keepdims=True)
    i2 = jnp.min(jnp.where(masked == m2, idxs, n_experts), axis=0,
                 keepdims=True)
    r = jnp.exp(m2 - m1)
    i1_ref[...] = i1[0]
    i2_ref[...] = i2[0]
    g1_ref[...] = (1.0 / (1.0 + r))[0]
    g2_ref[...] = (r / (1.0 + r))[0]
    pe = _pos_enc(x_ref[...])
    pe_ref[...] = jnp.concatenate(
        [pe, jnp.zeros((bm, 128 - 2 * ENC), jnp.float32)], axis=1)
    o1 = (i1 == idxs).astype(jnp.int32)
    o2 = (i2 == idxs).astype(jnp.int32)
    rows = []
    for hh in range(bm // 256):
        sl = slice(hh * 256, (hh + 1) * 256)
        rows.append(jnp.sum(o1[:, sl], axis=1)[None])
        rows.append(jnp.sum(o2[:, sl], axis=1)[None])
    h_ref[...] = jnp.concatenate(rows, axis=0)[None]


def _gating(x, gW0, gb0, gW1, gb1, n, n_experts):
    nblk = n // GBM
    nh = 2 * (GBM // 256)
    full = lambda a: pl.BlockSpec(a.shape, lambda m: (0,) * a.ndim)
    return pl.pallas_call(
        functools.partial(_gating_body, n_experts=n_experts),
        grid=(nblk,),
        in_specs=[
            pl.BlockSpec((2, GBM), lambda m: (0, m)),
            pl.BlockSpec((GBM, 2), lambda m: (m, 0)),
            full(gW0), full(gb0), full(gW1), full(gb1),
        ],
        out_specs=[
            pl.BlockSpec((GBM,), lambda m: (m,)),
            pl.BlockSpec((GBM,), lambda m: (m,)),
            pl.BlockSpec((GBM,), lambda m: (m,)),
            pl.BlockSpec((GBM,), lambda m: (m,)),
            pl.BlockSpec((GBM, 128), lambda m: (m, 0)),
            pl.BlockSpec((1, nh, n_experts), lambda m: (m, 0, 0)),
        ],
        out_shape=[
            jax.ShapeDtypeStruct((n,), jnp.int32),
            jax.ShapeDtypeStruct((n,), jnp.int32),
            jax.ShapeDtypeStruct((n,), jnp.float32),
            jax.ShapeDtypeStruct((n,), jnp.float32),
            jax.ShapeDtypeStruct((n, 128), jnp.float32),
            jax.ShapeDtypeStruct((nblk, nh, n_experts), jnp.int32),
        ],
        compiler_params=pltpu.CompilerParams(
            dimension_semantics=("arbitrary",),
        ),
    )(x.T, x, gW0, gb0, gW1, gb1)


def _dispatch(i1, i2, g1, g2, pe, hist, *, n, n_experts, pmax, nbp):
    pairs = 2 * n
    nw = 32
    chunk = pairs // nw
    nvec = chunk // 16
    mesh = plsc.VectorSubcoreMesh(core_axis_name="c", subcore_axis_name="s")

    @functools.partial(
        pl.kernel,
        out_type=(
            jax.ShapeDtypeStruct((pmax, 128), jnp.float32),
            jax.ShapeDtypeStruct((pairs // 128, 128), jnp.int32),
            jax.ShapeDtypeStruct((nbp,), jnp.int32),
        ),
        mesh=mesh,
        scratch_types=[
            pltpu.VMEM((chunk,), jnp.int32),
            pltpu.VMEM((chunk,), jnp.float32),
            pltpu.VMEM((nw * n_experts,), jnp.int32),
            pltpu.VMEM((chunk, 128), jnp.float32),
            pltpu.VMEM((2, 128), jnp.int32),
            pltpu.VMEM((nbp,), jnp.int32),
            pltpu.SemaphoreType.DMA,
        ],
        compiler_params=pltpu.CompilerParams(needs_layout_passes=False),
    )
    def dispatch(i1_hbm, i2_hbm, g1_hbm, g2_hbm, pe_hbm, hist_hbm,
                 xpad_hbm, pos_hbm, be_hbm,
                 ids_v, g_v, hist_v, bufg_v, pos_v, be_v, sem):
        wid = lax.axis_index("c") * 16 + lax.axis_index("s")
        s = wid % 2
        mb = wid // 2
        tok0 = mb * chunk

        @pl.when(s == 0)
        def _():
            pltpu.sync_copy(i1_hbm.at[pl.ds(tok0, chunk)], ids_v)
            pltpu.sync_copy(g1_hbm.at[pl.ds(tok0, chunk)], g_v)

        @pl.when(s == 1)
        def _():
            pltpu.sync_copy(i2_hbm.at[pl.ds(tok0, chunk)], ids_v)
            pltpu.sync_copy(g2_hbm.at[pl.ds(tok0, chunk)], g_v)

        pltpu.sync_copy(pe_hbm.at[pl.ds(tok0, chunk)], bufg_v)
        pltpu.sync_copy(hist_hbm, hist_v)

        iota = lax.iota(jnp.int32, 16)
        base = []
        po_next = []
        po_acc = jnp.int32(0)
        for e in range(n_experts):
            h0 = plsc.load_gather(hist_v, [iota * n_experts + e])
            h1 = plsc.load_gather(hist_v, [(iota + 16) * n_experts + e])
            tc_e = jnp.sum(h0) + jnp.sum(h1)
            prior = (jnp.sum(jnp.where(iota < wid, h0, 0))
                     + jnp.sum(jnp.where(iota + 16 < wid, h1, 0)))
            pc_e = jnp.left_shift(jnp.right_shift(tc_e + BM - 1, _SH), _SH)
            base.append(po_acc + prior)
            po_acc = po_acc + pc_e
            po_next.append(po_acc)

        @pl.when(wid == 0)
        def _():
            for i in range(nbp // 16):
                bi = iota + 16 * i
                bev = jnp.zeros((16,), jnp.int32)
                for e in range(n_experts):
                    nb_e = jnp.right_shift(po_next[e], _SH)
                    bev = bev + (bi >= nb_e).astype(jnp.int32)
                be_v[pl.ds(16 * i, 16)] = jnp.minimum(bev, n_experts - 1)
            pltpu.sync_copy(be_v, be_hbm)

        rb = list(base)
        zeros16 = jnp.zeros((16,), jnp.int32)
        for i in range(nvec):
            ev = ids_v[pl.ds(16 * i, 16)]
            gv = g_v[pl.ds(16 * i, 16)]
            pos = zeros16
            for e in range(n_experts):
                m = ev == e
                mi = m.astype(jnp.int32)
                cs = plsc.cumsum(mi)
                pos = jnp.where(m, rb[e] + cs - mi, pos)
                rb[e] = rb[e] + jnp.sum(mi)
            pos_v[i // 8, pl.ds((i % 8) * 16, 16)] = pos
            plsc.store_scatter(bufg_v, [iota + 16 * i, zeros16 + 16], gv)

        for h in range(2):
            src = bufg_v.at[pl.ds(h * 128, 128)]
            pltpu.async_copy(src, xpad_hbm.at[pos_v.at[h]], sem).wait()
        pltpu.sync_copy(pos_v, pos_hbm.at[pl.ds(s * (n // 128) + mb * 2, 2)])

    return dispatch(i1, i2, g1, g2, pe, hist)


def _group_body(be_ref, x_ref,
                SW0_ref, Sb0_ref, SW1_ref, Sb1_ref, SW2_ref, Sb2_ref,
                AW0_ref, Ab0_ref, AW1_ref, Ab1_ref, AW2_ref, Ab2_ref,
                out_ref):
    b = pl.program_id(0)
    e = be_ref[b]
    xg = x_ref[...]
    pe = xg[:, 0:16]
    gate = xg[:, 16:17]
    sw0 = SW0_ref[pl.ds(e, 1)][0]
    sb0 = Sb0_ref[pl.ds(e, 1)][0]
    sw1 = SW1_ref[pl.ds(e, 1)][0]
    sb1 = Sb1_ref[pl.ds(e, 1)][0]
    sw2 = SW2_ref[pl.ds(e, 1)][0]
    sb2 = Sb2_ref[pl.ds(e, 1)][0]
    aw0 = AW0_ref[pl.ds(e, 1)][0]
    ab0 = Ab0_ref[pl.ds(e, 1)][0]
    aw1 = AW1_ref[pl.ds(e, 1)][0]
    ab1 = Ab1_ref[pl.ds(e, 1)][0]
    aw2 = AW2_ref[0]
    ab2 = Ab2_ref[pl.ds(e, 1)][0]
    s = jax.nn.relu(jnp.dot(pe, sw0, preferred_element_type=jnp.float32) + sb0)
    s = jax.nn.relu(jnp.dot(s, sw1, preferred_element_type=jnp.float32) + sb1)
    iso = jnp.dot(s, sw2, preferred_element_type=jnp.float32) + sb2
    a = jax.nn.relu(iso * aw0[0][None, :] + ab0)
    a = jax.nn.relu(jnp.dot(a, aw1, preferred_element_type=jnp.float32) + ab1)
    o = jnp.dot(a, aw2, preferred_element_type=jnp.float32) + ab2
    out_ref[...] = gate * o


def _grouped(be, xpad, SW0, Sb0, SW1, Sb1, SW2, Sb2,
             AW0, Ab0, AW1, Ab1, AW2, Ab2, *, pmax, g_out):
    nb = pmax // BM
    Sb0, Sb1, Sb2 = Sb0[:, None], Sb1[:, None], Sb2[:, None]
    Ab0, Ab1, Ab2 = Ab0[:, None], Ab1[:, None], Ab2[:, None]
    wspec = lambda a: pl.BlockSpec(a.shape, lambda m, be: (0,) * a.ndim)
    aw2spec = pl.BlockSpec((1,) + AW2.shape[1:],
                           lambda m, be: (be[m], 0, 0))
    return pl.pallas_call(
        _group_body,
        grid_spec=pltpu.PrefetchScalarGridSpec(
            num_scalar_prefetch=1,
            grid=(nb,),
            in_specs=[
                pl.BlockSpec((BM, 128), lambda m, be: (m, 0)),
                wspec(SW0), wspec(Sb0), wspec(SW1), wspec(Sb1),
                wspec(SW2), wspec(Sb2),
                wspec(AW0), wspec(Ab0), wspec(AW1), wspec(Ab1),
                aw2spec, wspec(Ab2),
            ],
            out_specs=pl.BlockSpec((BM, g_out), lambda m, be: (m, 0)),
        ),
        out_shape=jax.ShapeDtypeStruct((pmax, g_out), jnp.float32),
        compiler_params=pltpu.CompilerParams(
            dimension_semantics=("arbitrary",),
        ),
    )(be, xpad, SW0, Sb0, SW1, Sb1, SW2, Sb2,
      AW0, Ab0, AW1, Ab1, AW2, Ab2)


def _combine(outpad, pos, *, n, g_out):
    nw = 32
    tpw = n // nw
    sub = 16
    nsub = tpw // sub
    mesh = plsc.VectorSubcoreMesh(core_axis_name="c", subcore_axis_name="s")

    @functools.partial(
        pl.kernel,
        out_type=jax.ShapeDtypeStruct((n, g_out), jnp.float32),
        mesh=mesh,
        scratch_types=[
            pltpu.VMEM((tpw,), jnp.int32),
            pltpu.VMEM((tpw,), jnp.int32),
            pltpu.VMEM((sub, g_out), jnp.float32),
            pltpu.VMEM((sub, g_out), jnp.float32),
            pltpu.VMEM((sub, g_out), jnp.float32),
            pltpu.VMEM((sub, g_out), jnp.float32),
            pltpu.SemaphoreType.DMA,
            pltpu.SemaphoreType.DMA,
            pltpu.SemaphoreType.DMA,
            pltpu.SemaphoreType.DMA,
        ],
        compiler_params=pltpu.CompilerParams(needs_layout_passes=False),
    )
    def combine(outpad_hbm, pos_hbm, y_hbm, p1_v, p2_v,
                rA1, rA2, rB1, rB2, sA, sB, swA, swB):
        wid = lax.axis_index("c") * 16 + lax.axis_index("s")
        pltpu.sync_copy(pos_hbm.at[wid], p1_v)
        pltpu.sync_copy(pos_hbm.at[nw + wid], p2_v)
        bufs = [(rA1, rA2, sA, swA), (rB1, rB2, sB, swB)]

        def gathers(k, i):
            r1b, r2b, sem, _ = bufs[i]
            c1 = pltpu.async_copy(
                outpad_hbm.at[p1_v.at[pl.ds(k * sub, sub)]], r1b, sem)
            c2 = pltpu.async_copy(
                outpad_hbm.at[p2_v.at[pl.ds(k * sub, sub)]], r2b, sem)
            return c1, c2

        pend = [None, None]
        wb = [None, None]
        pend[0] = gathers(0, 0)
        for k in range(nsub):
            i = k % 2
            j = (k + 1) % 2
            if k + 1 < nsub:
                if wb[j] is not None:
                    wb[j].wait()
                    wb[j] = None
                pend[j] = gathers(k + 1, j)
            c1, c2 = pend[i]
            c1.wait()
            c2.wait()
            r1b, r2b = bufs[i][0], bufs[i][1]

            @plsc.parallel_loop(0, sub * g_out, 64, unroll=4)
            def _(kk):
                r = kk // g_out
                o = kk % g_out
                for u in range(4):
                    plsc.addupdate(r1b.at[r, pl.ds(o + u * 16, 16)],
                                   r2b[r, pl.ds(o + u * 16, 16)])

            wb[i] = pltpu.async_copy(
                r1b, y_hbm.at[pl.ds(wid * tpw + k * sub, sub)], bufs[i][3])
        for i in range(2):
            if wb[i] is not None:
                wb[i].wait()

    return combine(outpad, pos)


def kernel(x, gW0, gb0, gW1, gb1, SW0, Sb0, SW1, Sb1, SW2, Sb2,
           AW0, Ab0, AW1, Ab1, AW2, Ab2):
    n = x.shape[0]
    n_experts = SW0.shape[0]
    g_out = AW2.shape[2]
    pmax = 2 * n + n_experts * BM
    nbp = ((pmax // BM) + 15) // 16 * 16

    i1, i2, g1, g2, pe, hist = _gating(x, gW0, gb0, gW1, gb1, n, n_experts)

    xpad, pos, be = _dispatch(i1, i2, g1, g2, pe, hist.reshape(-1),
                              n=n, n_experts=n_experts,
                              pmax=pmax, nbp=nbp)
    outpad = _grouped(be, xpad, SW0, Sb0, SW1, Sb1, SW2, Sb2,
                      AW0, Ab0, AW1, Ab1, AW2, Ab2,
                      pmax=pmax, g_out=g_out)
    y = _combine(outpad, pos, n=n, g_out=g_out)
    return y
